# Initial kernel scaffold; baseline (speedup 1.0000x reference)
import functools

import jax
import jax.numpy as jnp
from jax import lax
from jax.experimental import pallas as pl
from jax.experimental.pallas import tpu as pltpu
from jax.experimental.pallas import tpu_sc as plsc

F = 50000
L = 10000
T = 8
P = 32
H = 16

_HI = lax.Precision.HIGHEST

B_LP = 401408
CH_LP = 3136
B_PL = L * P
CH_PL = 2000


def _mm(a, b):
    return lax.dot_general(a, b, (((a.ndim - 1,), (0,)), ((), ())),
                           precision=_HI, preferred_element_type=jnp.float32)


def _sigmoid(x):
    return 1.0 / (1.0 + jnp.exp(-x))


def _selu(x):
    scale = 1.0507009873554804934193349852946
    alpha = 1.6732632423543772848170429916717
    return scale * jnp.where(x > 0, x, alpha * jnp.expm1(x))


def _gelu(x):
    return 0.5 * x * (1.0 + lax.erf(x * 0.7071067811865476))


def _softplus(x):
    return jnp.maximum(x, 0.0) + jnp.log1p(jnp.exp(-jnp.abs(x)))


def _sc_info():
    try:
        info = plsc.get_sparse_core_info()
        return info.num_cores, info.num_subcores
    except Exception:
        return 2, 16


@functools.lru_cache(maxsize=None)
def _rows_gather(N, D, B, chunk):
    NC, NS = _sc_info()
    NW = NC * NS
    bpw = B // NW
    nck = bpw // chunk
    assert B % NW == 0 and bpw % chunk == 0 and chunk % 8 == 0
    mesh = plsc.VectorSubcoreMesh(core_axis_name="c", subcore_axis_name="s")

    @functools.partial(
        pl.kernel, mesh=mesh,
        out_type=jax.ShapeDtypeStruct((B, D), jnp.float32),
        scratch_types=[
            pltpu.VMEM((chunk,), jnp.int32),
            pltpu.VMEM((chunk, D), jnp.float32),
            pltpu.SemaphoreType.DMA,
        ],
    )
    def g(table_hbm, idx_hbm, out_hbm, idx_v, rows_v, sem):
        wid = lax.axis_index("s") * NC + lax.axis_index("c")
        base = wid * bpw
        for c in range(nck):
            off = base + c * chunk
            pltpu.sync_copy(idx_hbm.at[pl.ds(off, chunk)], idx_v)
            pltpu.async_copy(table_hbm.at[idx_v], rows_v, sem).wait()
            pltpu.sync_copy(rows_v, out_hbm.at[pl.ds(off, chunk)])

    return g


@functools.lru_cache(maxsize=None)
def _scalar_gather(N, B, chunk):
    NC, NS = _sc_info()
    NW = NC * NS
    bpw = B // NW
    nck = bpw // chunk
    assert B % NW == 0 and bpw % chunk == 0 and chunk % 16 == 0
    mesh = plsc.VectorSubcoreMesh(core_axis_name="c", subcore_axis_name="s")

    @functools.partial(
        pl.kernel, mesh=mesh,
        out_type=jax.ShapeDtypeStruct((B,), jnp.float32),
        scratch_types=[
            pltpu.VMEM((N,), jnp.float32),
            pltpu.VMEM((chunk,), jnp.int32),
            pltpu.VMEM((chunk,), jnp.float32),
        ],
    )
    def g(table_hbm, idx_hbm, out_hbm, tab_v, idx_v, val_v):
        wid = lax.axis_index("s") * NC + lax.axis_index("c")
        base = wid * bpw
        pltpu.sync_copy(table_hbm, tab_v)
        for c in range(nck):
            off = base + c * chunk
            pltpu.sync_copy(idx_hbm.at[pl.ds(off, chunk)], idx_v)

            def body(i, carry):
                ii = idx_v[pl.ds(i * 16, 16)]
                val_v[pl.ds(i * 16, 16)] = plsc.load_gather(tab_v, [ii])
                return carry

            lax.fori_loop(0, chunk // 16, body, 0)
            pltpu.sync_copy(val_v, out_hbm.at[pl.ds(off, chunk)])

    return g


def _path_enc_body(ff, w1, b1, w2, b2, out):
    h1 = _selu(_mm(ff[...], w1[...]) + b1[...])
    out[...] = _selu(_mm(h1, w2[...]) + b2[...])


def _link_enc_body(cap, cn, ts, mll, w1, b1, w2, b2, out):
    load = jnp.sum(ts[...], axis=1, keepdims=True) / (cap[...] * 1e9)
    nload = load / mll[0, 0]
    lf = jnp.concatenate([cn[...][:, 0:1], load, nload], axis=1)
    h1 = _selu(_mm(lf, w1[...]) + b1[...])
    out[...] = _selu(_mm(h1, w2[...]) + b2[...])


def _path_gru_body(lg, ps, init, khat, r, bi, bh, ps_out, seq):
    x_all = _mm(lg[...], khat[...]) + bi[...]
    h = ps[...]
    seq[:, 0:H] = h + init[...]
    for t in range(T):
        gi = x_all[:, t * 48:(t + 1) * 48]
        gh = _mm(h, r[...]) + bh[...]
        z = _sigmoid(gi[:, 0:16] + gh[:, 0:16])
        rr = _sigmoid(gi[:, 16:32] + gh[:, 16:32])
        n = jnp.tanh(gi[:, 32:48] + rr * gh[:, 32:48])
        h = z * h + (1.0 - z) * n
        seq[:, (t + 1) * H:(t + 2) * H] = h
    ps_out[...] = h


def _att_link_body(pg, ls, init, ahat, ab, m, s, k, r, bi, bh, out):
    x = pg[...]
    coef = _mm(x, ahat[...]) + ab[...]
    coef = jnp.maximum(coef, 0.01 * coef)
    rowmax = jnp.max(coef, axis=1, keepdims=True)
    e = jnp.exp(coef - rowmax)
    denom = jnp.maximum(_mm(e, m[...]), 1e-30)
    score = _mm((e / denom) * x, s[...])
    h = ls[...] + init[...]
    gi = _mm(score, k[...]) + bi[...]
    gh = _mm(h, r[...]) + bh[...]
    z = _sigmoid(gi[:, 0:16] + gh[:, 0:16])
    rr = _sigmoid(gi[:, 16:32] + gh[:, 16:32])
    n = jnp.tanh(gi[:, 32:48] + rr * gh[:, 32:48])
    out[...] = z * h + (1.0 - z) * n


def _readout_body(seq, capg, w1, b1, w2, b2, w3, b3, out):
    x = seq[...][:, H:]
    h1 = _gelu(_mm(x, w1[...]) + b1[...])
    h2 = _gelu(_mm(h1, w2[...]) + b2[...])
    occ = _softplus(_mm(h2, w3[...]) + b3[...])
    out[...] = jnp.sum(occ / capg[...], axis=1, keepdims=True)


def _full(shape):
    return pl.BlockSpec(shape, lambda i: (0,) * len(shape))


def _rows(bs, width):
    return pl.BlockSpec((bs, width), lambda i: (i, 0))


def kernel(flow_traffic, flow_packets, global_delay, global_losses,
           max_link_load, flow_pkts_per_burst, flow_bitrate_per_burst,
           flow_packet_size, flow_type, flow_ipg_mean, ibg, flow_p90PktSize,
           rate, flow_ipg_var, link_capacity, link_capacity_and_node_type,
           fe_W1, fe_b1, fe_W2, fe_b2, le_W1, le_b1, le_W2, le_b2, att_W,
           att_b, pg_K, pg_R, pg_bi, pg_bh, lg_K, lg_R, lg_bi, lg_bh, ro_W1,
           ro_b1, ro_W2, ro_b2, ro_W3, ro_b3, flow_length, link_to_path,
           path_to_link):
    f32 = jnp.float32

    flow_feats = jnp.concatenate([
        flow_traffic, flow_packets, ibg, rate, flow_p90PktSize,
        flow_packet_size, flow_bitrate_per_burst, flow_ipg_mean,
        flow_ipg_var, flow_pkts_per_burst,
        flow_length.astype(f32)[:, None], flow_type], axis=1)

    p0 = path_to_link[:, :, 0].astype(jnp.int32)
    p1 = path_to_link[:, :, 1].astype(jnp.int32)
    idx_pl = (p0 * (T + 1) + p1).reshape(B_PL)
    idx_p0 = p0.reshape(B_PL)
    idx_lp = jnp.concatenate([
        link_to_path.astype(jnp.int32).reshape(F * T),
        jnp.zeros((B_LP - F * T,), jnp.int32)])

    eyeT = jnp.eye(T, dtype=f32)
    khat = jnp.kron(eyeT, pg_K)
    bi_t = jnp.tile(pg_bi, T)[None]
    ahat = jnp.kron(jnp.eye(P, dtype=f32), att_W)
    ab_t = jnp.tile(att_b, P)[None]
    m_mat = jnp.kron(jnp.eye(P, dtype=f32), jnp.ones((H, H), f32))
    s_mat = jnp.tile(jnp.eye(H, dtype=f32), (P, 1))
    w1hat = jnp.kron(eyeT, ro_W1)
    w2hat = jnp.kron(eyeT, ro_W2)
    w3hat = jnp.kron(eyeT, ro_W3)
    rb1 = jnp.tile(ro_b1, T)[None]
    rb2 = jnp.tile(ro_b2, T)[None]
    rb3 = jnp.tile(ro_b3, T)[None]

    ts = _scalar_gather(F, B_PL, CH_PL)(
        flow_traffic.reshape(F), idx_p0).reshape(L, P)

    BF = 2500
    nF = F // BF
    path_state = pl.pallas_call(
        _path_enc_body,
        grid=(nF,),
        in_specs=[_rows(BF, 13), _full((13, H)), _full((1, H)),
                  _full((H, H)), _full((1, H))],
        out_specs=_rows(BF, H),
        out_shape=jax.ShapeDtypeStruct((F, H), f32),
    )(flow_feats, fe_W1, fe_b1[None], fe_W2, fe_b2[None])

    link_state = pl.pallas_call(
        _link_enc_body,
        grid=(1,),
        in_specs=[_rows(L, 1), _rows(L, 3), _rows(L, P), _full((1, 1)),
                  _full((3, H)), _full((1, H)), _full((H, H)),
                  _full((1, H))],
        out_specs=_rows(L, H),
        out_shape=jax.ShapeDtypeStruct((L, H), f32),
    )(link_capacity, link_capacity_and_node_type, ts,
      max_link_load.reshape(1, 1), le_W1, le_b1[None], le_W2, le_b2[None])

    init_path = path_state
    init_link = link_state

    g1 = _rows_gather(L, H, B_LP, CH_LP)
    g2 = _rows_gather(F * (T + 1), H, B_PL, CH_PL)

    BL = 2500
    nL = L // BL
    seq = None
    for _ in range(12):
        lg = g1(link_state, idx_lp).reshape(B_LP // T, T * H)

        path_state, seq = pl.pallas_call(
            _path_gru_body,
            grid=(nF,),
            in_specs=[_rows(BF, T * H), _rows(BF, H), _rows(BF, H),
                      _full((T * H, 3 * H * T)), _full((H, 3 * H)),
                      _full((1, 3 * H * T)), _full((1, 3 * H))],
            out_specs=[_rows(BF, H), _rows(BF, (T + 1) * H)],
            out_shape=[jax.ShapeDtypeStruct((F, H), f32),
                       jax.ShapeDtypeStruct((F, (T + 1) * H), f32)],
        )(lg, path_state, init_path, khat, pg_R, bi_t, pg_bh[None])

        pg = g2(seq.reshape(F * (T + 1), H), idx_pl).reshape(L, P * H)

        link_state = pl.pallas_call(
            _att_link_body,
            grid=(nL,),
            in_specs=[_rows(BL, P * H), _rows(BL, H), _rows(BL, H),
                      _full((P * H, P * H)), _full((1, P * H)),
                      _full((P * H, P * H)), _full((P * H, H)),
                      _full((H, 3 * H)), _full((H, 3 * H)),
                      _full((1, 3 * H)), _full((1, 3 * H))],
            out_specs=_rows(BL, H),
            out_shape=jax.ShapeDtypeStruct((L, H), f32),
        )(pg, link_state, init_link, ahat, ab_t, m_mat, s_mat,
          lg_K, lg_R, lg_bi[None], lg_bh[None])

    capg = _scalar_gather(L, B_LP, CH_LP)(
        link_capacity.reshape(L), idx_lp).reshape(B_LP // T, T)

    delay = pl.pallas_call(
        _readout_body,
        grid=(nF,),
        in_specs=[_rows(BF, (T + 1) * H), _rows(BF, T), _full((T * H, 64)),
                  _full((1, 64)), _full((64, 32)), _full((1, 32)),
                  _full((32, 8)), _full((1, 8))],
        out_specs=_rows(BF, 1),
        out_shape=jax.ShapeDtypeStruct((F, 1), f32),
    )(seq, capg, w1hat, rb1, w2hat, rb2, w3hat, rb3)

    return delay


# speedup vs baseline: 1.6510x; 1.6510x over previous
import functools

import jax
import jax.numpy as jnp
from jax import lax
from jax.experimental import pallas as pl
from jax.experimental.pallas import tpu as pltpu
from jax.experimental.pallas import tpu_sc as plsc

F = 50000
L = 10000
T = 8
P = 32
H = 16

_HI = lax.Precision.HIGHEST

B_LP = 401408
CH_LP = 3136
B_PL = L * P
CH_PL = 2000


def _mm(a, b):
    return lax.dot_general(a, b, (((a.ndim - 1,), (0,)), ((), ())),
                           precision=_HI, preferred_element_type=jnp.float32)


def _sigmoid(x):
    return 1.0 / (1.0 + jnp.exp(-x))


def _selu(x):
    scale = 1.0507009873554804934193349852946
    alpha = 1.6732632423543772848170429916717
    return scale * jnp.where(x > 0, x, alpha * (jnp.exp(jnp.minimum(x, 0.0)) - 1.0))


def _gelu(x):
    return 0.5 * x * (1.0 + lax.erf(x * 0.7071067811865476))


def _softplus(x):
    return jnp.maximum(x, 0.0) + jnp.log(1.0 + jnp.exp(-jnp.abs(x)))


def _sc_info():
    try:
        info = plsc.get_sparse_core_info()
        return info.num_cores, info.num_subcores
    except Exception:
        return 2, 16


@functools.lru_cache(maxsize=None)
def _rows_gather(N, D, B, chunk):
    NC, NS = _sc_info()
    NW = NC * NS
    bpw = B // NW
    nck = bpw // chunk
    assert B % NW == 0 and bpw % chunk == 0 and chunk % 8 == 0
    mesh = plsc.VectorSubcoreMesh(core_axis_name="c", subcore_axis_name="s")

    @functools.partial(
        pl.kernel, mesh=mesh,
        out_type=jax.ShapeDtypeStruct((B, D), jnp.float32),
        compiler_params=pltpu.CompilerParams(use_tc_tiling_on_sc=False),
        scratch_types=[
            pltpu.VMEM((chunk,), jnp.int32),
            pltpu.VMEM((chunk, D), jnp.float32),
            pltpu.SemaphoreType.DMA,
        ],
    )
    def g(table_hbm, idx_hbm, out_hbm, idx_v, rows_v, sem):
        wid = lax.axis_index("s") * NC + lax.axis_index("c")
        base = wid * bpw
        for c in range(nck):
            off = base + c * chunk
            pltpu.sync_copy(idx_hbm.at[pl.ds(off, chunk)], idx_v)
            pltpu.async_copy(table_hbm.at[idx_v], rows_v, sem).wait()
            pltpu.sync_copy(rows_v, out_hbm.at[pl.ds(off, chunk)])

    return g


def _path_enc_body(ff, w1, b1, w2, b2, out):
    h1 = _selu(_mm(ff[...], w1[...]) + b1[...])
    out[...] = _selu(_mm(h1, w2[...]) + b2[...])


def _link_enc_body(cap, cn, ts, mll, w1, b1, w2, b2, out):
    load = jnp.sum(ts[...], axis=1, keepdims=True) / (cap[...] * 1e9)
    nload = load / mll[0, 0]
    lf = jnp.concatenate([cn[...][:, 0:1], load, nload], axis=1)
    h1 = _selu(_mm(lf, w1[...]) + b1[...])
    out[...] = _selu(_mm(h1, w2[...]) + b2[...])


def _path_gru_body(lg, ps, init, khat, r, bi, bh, ps_out, seq):
    x_all = _mm(lg[...], khat[...]) + bi[...]
    h = ps[...]
    seq[:, 0:H] = h + init[...]
    for t in range(T):
        gi = x_all[:, t * 48:(t + 1) * 48]
        gh = _mm(h, r[...]) + bh[...]
        z = _sigmoid(gi[:, 0:16] + gh[:, 0:16])
        rr = _sigmoid(gi[:, 16:32] + gh[:, 16:32])
        n = jnp.tanh(gi[:, 32:48] + rr * gh[:, 32:48])
        h = z * h + (1.0 - z) * n
        seq[:, (t + 1) * H:(t + 2) * H] = h
    ps_out[...] = h


def _att_link_body(pg, ls, init, ahat, ab, m, s, k, r, bi, bh, out):
    x = pg[...]
    coef = _mm(x, ahat[...]) + ab[...]
    coef = jnp.maximum(coef, 0.01 * coef)
    rowmax = jnp.max(coef, axis=1, keepdims=True)
    e = jnp.exp(coef - rowmax)
    denom = jnp.maximum(_mm(e, m[...]), 1e-30)
    score = _mm((e / denom) * x, s[...])
    h = ls[...] + init[...]
    gi = _mm(score, k[...]) + bi[...]
    gh = _mm(h, r[...]) + bh[...]
    z = _sigmoid(gi[:, 0:16] + gh[:, 0:16])
    rr = _sigmoid(gi[:, 16:32] + gh[:, 16:32])
    n = jnp.tanh(gi[:, 32:48] + rr * gh[:, 32:48])
    out[...] = z * h + (1.0 - z) * n


def _readout_body(seq, capg, w1, b1, w2, b2, w3, b3, out):
    x = seq[...][:, H:]
    h1 = _gelu(_mm(x, w1[...]) + b1[...])
    h2 = _gelu(_mm(h1, w2[...]) + b2[...])
    occ = _softplus(_mm(h2, w3[...]) + b3[...])
    out[...] = jnp.sum(occ / capg[...], axis=1, keepdims=True)


def _full(shape):
    return pl.BlockSpec(shape, lambda i: (0,) * len(shape))


def _rows(bs, width):
    return pl.BlockSpec((bs, width), lambda i: (i, 0))


def kernel(flow_traffic, flow_packets, global_delay, global_losses,
           max_link_load, flow_pkts_per_burst, flow_bitrate_per_burst,
           flow_packet_size, flow_type, flow_ipg_mean, ibg, flow_p90PktSize,
           rate, flow_ipg_var, link_capacity, link_capacity_and_node_type,
           fe_W1, fe_b1, fe_W2, fe_b2, le_W1, le_b1, le_W2, le_b2, att_W,
           att_b, pg_K, pg_R, pg_bi, pg_bh, lg_K, lg_R, lg_bi, lg_bh, ro_W1,
           ro_b1, ro_W2, ro_b2, ro_W3, ro_b3, flow_length, link_to_path,
           path_to_link):
    f32 = jnp.float32

    flow_feats = jnp.concatenate([
        flow_traffic, flow_packets, ibg, rate, flow_p90PktSize,
        flow_packet_size, flow_bitrate_per_burst, flow_ipg_mean,
        flow_ipg_var, flow_pkts_per_burst,
        flow_length.astype(f32)[:, None], flow_type], axis=1)

    p0 = path_to_link[:, :, 0].astype(jnp.int32)
    p1 = path_to_link[:, :, 1].astype(jnp.int32)
    idx_pl = (p0 * (T + 1) + p1).reshape(B_PL)
    idx_p0 = p0.reshape(B_PL)
    idx_lp = jnp.concatenate([
        link_to_path.astype(jnp.int32).reshape(F * T),
        jnp.zeros((B_LP - F * T,), jnp.int32)])

    eyeT = jnp.eye(T, dtype=f32)
    khat = jnp.kron(eyeT, pg_K)
    bi_t = jnp.tile(pg_bi, T)[None]
    ahat = jnp.kron(jnp.eye(P, dtype=f32), att_W)
    ab_t = jnp.tile(att_b, P)[None]
    m_mat = jnp.kron(jnp.eye(P, dtype=f32), jnp.ones((H, H), f32))
    s_mat = jnp.tile(jnp.eye(H, dtype=f32), (P, 1))
    w1hat = jnp.kron(eyeT, ro_W1)
    w2hat = jnp.kron(eyeT, ro_W2)
    w3hat = jnp.kron(eyeT, ro_W3)
    rb1 = jnp.tile(ro_b1, T)[None]
    rb2 = jnp.tile(ro_b2, T)[None]
    rb3 = jnp.tile(ro_b3, T)[None]

    ts = _rows_gather(F, H, B_PL, CH_PL)(
        jnp.tile(flow_traffic, (1, H)), idx_p0)[:, 0].reshape(L, P)

    BF = 2000
    nF = F // BF
    path_state = pl.pallas_call(
        _path_enc_body,
        grid=(nF,),
        in_specs=[_rows(BF, 13), _full((13, H)), _full((1, H)),
                  _full((H, H)), _full((1, H))],
        out_specs=_rows(BF, H),
        out_shape=jax.ShapeDtypeStruct((F, H), f32),
    )(flow_feats, fe_W1, fe_b1[None], fe_W2, fe_b2[None])

    link_state = pl.pallas_call(
        _link_enc_body,
        grid=(1,),
        in_specs=[_rows(L, 1), _rows(L, 3), _rows(L, P), _full((1, 1)),
                  _full((3, H)), _full((1, H)), _full((H, H)),
                  _full((1, H))],
        out_specs=_rows(L, H),
        out_shape=jax.ShapeDtypeStruct((L, H), f32),
    )(link_capacity, link_capacity_and_node_type, ts,
      max_link_load.reshape(1, 1), le_W1, le_b1[None], le_W2, le_b2[None])

    init_path = path_state
    init_link = link_state

    g1 = _rows_gather(L, H, B_LP, CH_LP)
    g2 = _rows_gather(F * (T + 1), H, B_PL, CH_PL)

    BL = 2000
    nL = L // BL
    seq = None
    for _ in range(12):
        lg = g1(link_state, idx_lp).reshape(B_LP // T, T * H)

        path_state, seq = pl.pallas_call(
            _path_gru_body,
            grid=(nF,),
            in_specs=[_rows(BF, T * H), _rows(BF, H), _rows(BF, H),
                      _full((T * H, 3 * H * T)), _full((H, 3 * H)),
                      _full((1, 3 * H * T)), _full((1, 3 * H))],
            out_specs=[_rows(BF, H), _rows(BF, (T + 1) * H)],
            out_shape=[jax.ShapeDtypeStruct((F, H), f32),
                       jax.ShapeDtypeStruct((F, (T + 1) * H), f32)],
        )(lg, path_state, init_path, khat, pg_R, bi_t, pg_bh[None])

        pg = g2(seq.reshape(F * (T + 1), H), idx_pl).reshape(L, P * H)

        link_state = pl.pallas_call(
            _att_link_body,
            grid=(nL,),
            in_specs=[_rows(BL, P * H), _rows(BL, H), _rows(BL, H),
                      _full((P * H, P * H)), _full((1, P * H)),
                      _full((P * H, P * H)), _full((P * H, H)),
                      _full((H, 3 * H)), _full((H, 3 * H)),
                      _full((1, 3 * H)), _full((1, 3 * H))],
            out_specs=_rows(BL, H),
            out_shape=jax.ShapeDtypeStruct((L, H), f32),
        )(pg, link_state, init_link, ahat, ab_t, m_mat, s_mat,
          lg_K, lg_R, lg_bi[None], lg_bh[None])

    capg = _rows_gather(L, H, B_LP, CH_LP)(
        jnp.tile(link_capacity, (1, H)), idx_lp)[:, 0].reshape(B_LP // T, T)

    delay = pl.pallas_call(
        _readout_body,
        grid=(nF,),
        in_specs=[_rows(BF, (T + 1) * H), _rows(BF, T), _full((T * H, 64)),
                  _full((1, 64)), _full((64, 32)), _full((1, 32)),
                  _full((32, 8)), _full((1, 8))],
        out_specs=_rows(BF, 1),
        out_shape=jax.ShapeDtypeStruct((F, 1), f32),
    )(seq, capg, w1hat, rb1, w2hat, rb2, w3hat, rb3)

    return delay


# speedup vs baseline: 2.3306x; 1.4116x over previous
import functools

import jax
import jax.numpy as jnp
from jax import lax
from jax.experimental import pallas as pl
from jax.experimental.pallas import tpu as pltpu
from jax.experimental.pallas import tpu_sc as plsc

F = 50000
L = 10000
T = 8
P = 32
H = 16

_HI = lax.Precision.HIGHEST

B_LP = 401408
CH_LP = 3136
B_PL = L * P
CH_PL = 2000


def _mm(a, b):
    return lax.dot_general(a, b, (((a.ndim - 1,), (0,)), ((), ())),
                           precision=_HI, preferred_element_type=jnp.float32)


def _sigmoid(x):
    return 1.0 / (1.0 + jnp.exp(-x))


def _selu(x):
    scale = 1.0507009873554804934193349852946
    alpha = 1.6732632423543772848170429916717
    return scale * jnp.where(x > 0, x, alpha * (jnp.exp(jnp.minimum(x, 0.0)) - 1.0))


def _gelu(x):
    return 0.5 * x * (1.0 + lax.erf(x * 0.7071067811865476))


def _softplus(x):
    return jnp.maximum(x, 0.0) + jnp.log(1.0 + jnp.exp(-jnp.abs(x)))


def _sc_info():
    try:
        info = plsc.get_sparse_core_info()
        return info.num_cores, info.num_subcores
    except Exception:
        return 2, 16


@functools.lru_cache(maxsize=None)
def _rows_gather(N, D, B, chunk):
    NC, NS = _sc_info()
    NW = NC * NS
    bpw = B // NW
    nck = bpw // chunk
    assert B % NW == 0 and bpw % chunk == 0 and chunk % 8 == 0
    mesh = plsc.VectorSubcoreMesh(core_axis_name="c", subcore_axis_name="s")

    @functools.partial(
        pl.kernel, mesh=mesh,
        out_type=jax.ShapeDtypeStruct((B, D), jnp.float32),
        compiler_params=pltpu.CompilerParams(use_tc_tiling_on_sc=False),
        scratch_types=[
            pltpu.VMEM((chunk,), jnp.int32),
            pltpu.VMEM((chunk, D), jnp.float32),
            pltpu.SemaphoreType.DMA,
        ],
    )
    def g(table_hbm, idx_hbm, out_hbm, idx_v, rows_v, sem):
        wid = lax.axis_index("s") * NC + lax.axis_index("c")
        base = wid * bpw
        for c in range(nck):
            off = base + c * chunk
            pltpu.sync_copy(idx_hbm.at[pl.ds(off, chunk)], idx_v)
            pltpu.async_copy(table_hbm.at[idx_v], rows_v, sem).wait()
            pltpu.sync_copy(rows_v, out_hbm.at[pl.ds(off, chunk)])

    return g


def _path_enc_body(ff, w1, b1, w2, b2, out):
    h1 = _selu(_mm(ff[...], w1[...]) + b1[...])
    out[...] = _selu(_mm(h1, w2[...]) + b2[...])


def _link_enc_body(cap, cn, ts, mll, w1, b1, w2, b2, out):
    load = jnp.sum(ts[...], axis=1, keepdims=True) / (cap[...] * 1e9)
    nload = load / mll[0, 0]
    lf = jnp.concatenate([cn[...][:, 0:1], load, nload], axis=1)
    h1 = _selu(_mm(lf, w1[...]) + b1[...])
    out[...] = _selu(_mm(h1, w2[...]) + b2[...])


def _path_gru_body(lg, ps, init, kbig, rtil, bi, bh, ps_out, seq):
    x_all = _mm(lg[...], kbig[...]) + bi[...]
    h = ps[...]
    seq[0] = h + init[...]
    for t in range(T):
        xt = x_all[:, t * 384:(t + 1) * 384]
        gh = _mm(h, rtil[...]) + bh[...]
        z = _sigmoid(xt[:, 0:128] + gh[:, 0:128])
        rr = _sigmoid(xt[:, 128:256] + gh[:, 128:256])
        n = jnp.tanh(xt[:, 256:384] + rr * gh[:, 256:384])
        h = z * h + (1.0 - z) * n
        seq[t + 1] = h
    ps_out[...] = h


def _att_link_body(pg, ls, init, ahat, ab, m, s, k, r, bi, bh, out):
    x = pg[...]
    coef = _mm(x, ahat[...]) + ab[...]
    coef = jnp.maximum(coef, 0.01 * coef)
    rowmax = jnp.max(coef, axis=1, keepdims=True)
    e = jnp.exp(coef - rowmax)
    denom = jnp.maximum(_mm(e, m[...]), 1e-30)
    score = _mm((e / denom) * x, s[...])
    h = ls[...] + init[...]
    gi = _mm(score, k[...]) + bi[...]
    gh = _mm(h, r[...]) + bh[...]
    z = _sigmoid(gi[:, 0:16] + gh[:, 0:16])
    rr = _sigmoid(gi[:, 16:32] + gh[:, 16:32])
    n = jnp.tanh(gi[:, 32:48] + rr * gh[:, 32:48])
    out[...] = z * h + (1.0 - z) * n


def _readout_body(seq, capg, w1, b1, w2, b2, w3, b3, out):
    acc = jnp.zeros(out.shape, jnp.float32)
    for t in range(T):
        x = seq[t + 1]
        h1 = _gelu(_mm(x, w1[...]) + b1[...])
        h2 = _gelu(_mm(h1, w2[...]) + b2[...])
        occ = _softplus(_mm(h2, w3[...]) + b3[...])
        acc = acc + occ / capg[t]
    out[...] = acc


def _full(shape):
    return pl.BlockSpec(shape, lambda i: (0,) * len(shape))


def _rows(bs, width):
    return pl.BlockSpec((bs, width), lambda i: (i, 0))


def kernel(flow_traffic, flow_packets, global_delay, global_losses,
           max_link_load, flow_pkts_per_burst, flow_bitrate_per_burst,
           flow_packet_size, flow_type, flow_ipg_mean, ibg, flow_p90PktSize,
           rate, flow_ipg_var, link_capacity, link_capacity_and_node_type,
           fe_W1, fe_b1, fe_W2, fe_b2, le_W1, le_b1, le_W2, le_b2, att_W,
           att_b, pg_K, pg_R, pg_bi, pg_bh, lg_K, lg_R, lg_bi, lg_bh, ro_W1,
           ro_b1, ro_W2, ro_b2, ro_W3, ro_b3, flow_length, link_to_path,
           path_to_link):
    f32 = jnp.float32

    flow_feats = jnp.concatenate([
        flow_traffic, flow_packets, ibg, rate, flow_p90PktSize,
        flow_packet_size, flow_bitrate_per_burst, flow_ipg_mean,
        flow_ipg_var, flow_pkts_per_burst,
        flow_length.astype(f32)[:, None], flow_type], axis=1)

    FP = F // 8
    FPP = 6256
    FLP = FPP * 8
    BFP = 256
    nF = (FP + BFP - 1) // BFP

    p0 = path_to_link[:, :, 0].astype(jnp.int32)
    p1 = path_to_link[:, :, 1].astype(jnp.int32)
    idx_pl = (p1 * FLP + p0).reshape(B_PL)
    idx_p0 = p0.reshape(B_PL)
    idx_lp = jnp.concatenate([
        link_to_path.astype(jnp.int32).reshape(F * T),
        jnp.zeros((B_LP - F * T,), jnp.int32)])
    idx_lp_t = jnp.concatenate([
        link_to_path.astype(jnp.int32).T.reshape(T * F),
        jnp.zeros((B_LP - F * T,), jnp.int32)])

    eye8 = jnp.eye(8, dtype=f32)

    def pack_cols(w):
        return jnp.kron(eye8, w)

    def tile8(b):
        return jnp.tile(b, 8)[None]

    kcols = []
    for t in range(T):
        et = jnp.eye(8 * H, H, -H * t, dtype=f32)
        kcols.extend(
            pack_cols(et @ pg_K[:, g * H:(g + 1) * H]) for g in range(3))
    kbig = jnp.concatenate(kcols, axis=1)
    rtil = jnp.concatenate(
        [pack_cols(pg_R[:, g * H:(g + 1) * H]) for g in range(3)], axis=1)
    bi3 = jnp.concatenate(
        [jnp.tile(pg_bi[g * H:(g + 1) * H], 8) for g in range(3)])
    bi_t = jnp.tile(bi3, T)[None]
    bh3 = jnp.concatenate(
        [jnp.tile(pg_bh[g * H:(g + 1) * H], 8) for g in range(3)])[None]

    ahat = jnp.kron(jnp.eye(P, dtype=f32), att_W)
    ab_t = jnp.tile(att_b, P)[None]
    m_mat = jnp.kron(jnp.eye(P, dtype=f32), jnp.ones((H, H), f32))
    s_mat = jnp.tile(jnp.eye(H, dtype=f32), (P, 1))

    fe_w1p = pack_cols(fe_W1)
    fe_w2p = pack_cols(fe_W2)
    w1p = pack_cols(ro_W1)
    w2p = pack_cols(ro_W2)
    w3p = pack_cols(ro_W3)

    ts = _rows_gather(F, H, B_PL, CH_PL)(
        jnp.tile(flow_traffic, (1, H)), idx_p0)[:, 0].reshape(L, P)

    path_state = pl.pallas_call(
        _path_enc_body,
        grid=(nF,),
        in_specs=[_rows(BFP, 104), _full((104, 128)), _full((1, 128)),
                  _full((128, 128)), _full((1, 128))],
        out_specs=_rows(BFP, 128),
        out_shape=jax.ShapeDtypeStruct((FP, 128), f32),
    )(flow_feats.reshape(FP, 104), fe_w1p, tile8(fe_b1), fe_w2p,
      tile8(fe_b2))

    link_state = pl.pallas_call(
        _link_enc_body,
        grid=(1,),
        in_specs=[_rows(L, 1), _rows(L, 3), _rows(L, P), _full((1, 1)),
                  _full((3, H)), _full((1, H)), _full((H, H)),
                  _full((1, H))],
        out_specs=_rows(L, H),
        out_shape=jax.ShapeDtypeStruct((L, H), f32),
    )(link_capacity, link_capacity_and_node_type, ts,
      max_link_load.reshape(1, 1), le_W1, le_b1[None], le_W2, le_b2[None])

    init_path = path_state
    init_link = link_state

    g1 = _rows_gather(L, H, B_LP, CH_LP)
    g2 = _rows_gather(FLP * (T + 1), H, B_PL, CH_PL)

    BL = 2000
    nL = L // BL
    seq = None
    seq_spec = pl.BlockSpec((T + 1, BFP, 128), lambda i: (0, i, 0))
    for _ in range(12):
        lg = g1(link_state, idx_lp).reshape(B_LP // 64, 1024)

        path_state, seq = pl.pallas_call(
            _path_gru_body,
            grid=(nF,),
            in_specs=[_rows(BFP, 1024), _rows(BFP, 128), _rows(BFP, 128),
                      _full((1024, 3072)), _full((128, 384)),
                      _full((1, 3072)), _full((1, 384))],
            out_specs=[_rows(BFP, 128), seq_spec],
            out_shape=[jax.ShapeDtypeStruct((FP, 128), f32),
                       jax.ShapeDtypeStruct((T + 1, FPP, 128), f32)],
        )(lg, path_state, init_path, kbig, rtil, bi_t, bh3)

        pg = g2(seq.reshape(FLP * (T + 1), H), idx_pl).reshape(L, P * H)

        link_state = pl.pallas_call(
            _att_link_body,
            grid=(nL,),
            in_specs=[_rows(BL, P * H), _rows(BL, H), _rows(BL, H),
                      _full((P * H, P * H)), _full((1, P * H)),
                      _full((P * H, P * H)), _full((P * H, H)),
                      _full((H, 3 * H)), _full((H, 3 * H)),
                      _full((1, 3 * H)), _full((1, 3 * H))],
            out_specs=_rows(BL, H),
            out_shape=jax.ShapeDtypeStruct((L, H), f32),
        )(pg, link_state, init_link, ahat, ab_t, m_mat, s_mat,
          lg_K, lg_R, lg_bi[None], lg_bh[None])

    capg = _rows_gather(L, H, B_LP, CH_LP)(
        jnp.tile(link_capacity, (1, H)), idx_lp_t)[:, 0]
    capg = capg[:F * T].reshape(T, FP, 8)

    delay = pl.pallas_call(
        _readout_body,
        grid=(nF,),
        in_specs=[seq_spec, pl.BlockSpec((T, BFP, 8), lambda i: (0, i, 0)),
                  _full((128, 64)), _full((1, 64)), _full((64, 32)),
                  _full((1, 32)), _full((32, 8)), _full((1, 8))],
        out_specs=_rows(BFP, 8),
        out_shape=jax.ShapeDtypeStruct((FP, 8), f32),
    )(seq, capg, w1p, tile8(ro_b1), w2p, tile8(ro_b2), w3p, tile8(ro_b3))

    return delay.reshape(F, 1)
